# Initial kernel scaffold; baseline (speedup 1.0000x reference)
import jax
import jax.numpy as jnp
from jax.experimental import pallas as pl


def kernel(x, edge_index1, edge_weight1, edge_index2, edge_weight2, mapping, W1, b1, W2, b2, Wfc, bfc):
    raise NotImplementedError("write your pallas kernel here")


# speedup vs baseline: 33.5493x; 33.5493x over previous
import functools

import jax
import jax.numpy as jnp
from jax import lax
from jax.experimental import pallas as pl
from jax.experimental.pallas import tpu as pltpu
from jax.experimental.pallas import tpu_sc as plsc

KCH = 12
HW = 15
NG1 = 32
NG2 = 64
NN1 = 10000
N1P = 10240
NN2 = 1000
NT = 29
NB = 8
NC = 6

SC_CORES = 2
SC_TILES = 16
LANES = 16

F1 = 256
F1H = F1 // 2
CH = 128

N2PAD = 1024
N2COL = 1008


def _ceil_to(a, m):
  return (a + m - 1) // m * m


def _make_sc_prop(n_nodes, e_pad, first):
  cpt = e_pad // (SC_TILES * CH)
  rpt = n_nodes // SC_TILES
  rb = 32
  assert rpt % rb == 0

  mesh = plsc.VectorSubcoreMesh(core_axis_name="c", subcore_axis_name="s")

  def body(y_hbm, src_hbm, dst_hbm, w16_hbm, prev2_hbm, out_hbm,
           idx_v, dst_v, w16_v, rows_v, blk_v, prev_v, acc_sh, gsem):
    c = lax.axis_index("c")
    s = lax.axis_index("s")

    for i in range(rb):
      for f in range(F1H // LANES):
        blk_v[i, pl.ds(f * LANES, LANES)] = jnp.zeros((LANES,), jnp.float32)
    def zero_blk(t, _):
      pltpu.sync_copy(blk_v, acc_sh.at[pl.ds(s * rpt + t * rb, rb)])
      return 0
    lax.fori_loop(0, rpt // rb, zero_blk, 0)
    plsc.subcore_barrier()

    def chunk(j, _):
      base = (s * cpt + j) * CH
      pltpu.sync_copy(src_hbm.at[pl.ds(base, CH)], idx_v)
      pltpu.sync_copy(dst_hbm.at[pl.ds(base, CH)], dst_v)
      pltpu.sync_copy(w16_hbm.at[pl.ds(base, CH)], w16_v)
      off = c * n_nodes
      for i in range(CH // LANES):
        sl = pl.ds(i * LANES, LANES)
        idx_v[sl] = idx_v[sl] + off
      pltpu.async_copy(y_hbm.at[idx_v], rows_v, gsem).wait()
      for e in range(CH):
        wb = w16_v[e]
        for f in range(F1H // LANES):
          sl = pl.ds(f * LANES, LANES)
          rows_v[e, sl] = rows_v[e, sl] * wb
      pltpu.sync_copy(rows_v, acc_sh.at[dst_v], add=True)
      return 0
    lax.fori_loop(0, cpt, chunk, 0)
    plsc.subcore_barrier()

    def epi(t, _):
      r0 = s * rpt + t * rb
      pltpu.sync_copy(acc_sh.at[pl.ds(r0, rb)], blk_v)
      if not first:
        pltpu.sync_copy(prev2_hbm.at[pl.ds(c * n_nodes + r0, rb)], prev_v)
        for i in range(rb):
          for f in range(F1H // LANES):
            sl = pl.ds(f * LANES, LANES)
            a = blk_v[i, sl]
            blk_v[i, sl] = a + a - prev_v[i, sl]
      pltpu.sync_copy(blk_v, out_hbm.at[pl.ds(c * n_nodes + r0, rb)])
      return 0
    lax.fori_loop(0, rpt // rb, epi, 0)

  kern = pl.kernel(
      body,
      out_type=jax.ShapeDtypeStruct((2 * n_nodes, F1H), jnp.float32),
      mesh=mesh,
      scratch_types=[
          pltpu.VMEM((CH,), jnp.int32),
          pltpu.VMEM((CH,), jnp.int32),
          pltpu.VMEM((CH, LANES), jnp.float32),
          pltpu.VMEM((CH, F1H), jnp.float32),
          pltpu.VMEM((rb, F1H), jnp.float32),
          pltpu.VMEM((rb, F1H), jnp.float32),
          pltpu.VMEM_SHARED((n_nodes, F1H), jnp.float32),
          pltpu.SemaphoreType.DMA,
      ],
  )
  return kern


def _make_sc_dense_a2(e_pad):
  cpt = e_pad // (SC_TILES * CH)
  half = N2PAD // 2
  rpt = half // SC_TILES
  mesh = plsc.VectorSubcoreMesh(core_axis_name="c", subcore_axis_name="s")

  blk_rows = 8
  blk_words = blk_rows * N2COL

  def body(src_hbm, dst_hbm, w_hbm, out_hbm, idx_v, src_v, w_v, blk_v, a_sh):
    c = lax.axis_index("c")
    s = lax.axis_index("s")

    for i in range(blk_words // LANES):
      blk_v[pl.ds(i * LANES, LANES)] = jnp.zeros((LANES,), jnp.float32)
    def zero_blk(t, _):
      pltpu.sync_copy(blk_v, a_sh.at[pl.ds((s * rpt + t * blk_rows) * N2COL,
                                           blk_words)])
      return 0
    lax.fori_loop(0, rpt // blk_rows, zero_blk, 0)
    plsc.subcore_barrier()

    def chunk(j, _):
      base = (s * cpt + j) * CH
      pltpu.sync_copy(dst_hbm.at[pl.ds(base, CH)], idx_v)
      pltpu.sync_copy(src_hbm.at[pl.ds(base, CH)], src_v)
      pltpu.sync_copy(w_hbm.at[pl.ds(base, CH)], w_v)
      lo = c * half
      for i in range(CH // LANES):
        sl = pl.ds(i * LANES, LANES)
        d = idx_v[sl]
        ok = (d >= lo) & (d < lo + half)
        spread = lax.iota(jnp.int32, LANES) + i * LANES
        flat = (d - lo) * N2COL + src_v[sl]
        idx_v[sl] = jnp.where(ok, flat, spread)
        w_v[sl] = jnp.where(ok, w_v[sl], jnp.zeros((LANES,), jnp.float32))
      pltpu.sync_copy(w_v, a_sh.at[idx_v], add=True)
      return 0
    lax.fori_loop(0, cpt, chunk, 0)
    plsc.subcore_barrier()

    def wr(t, _):
      w0 = (s * rpt + t * blk_rows) * N2COL
      pltpu.sync_copy(a_sh.at[pl.ds(w0, blk_words)], blk_v)
      pltpu.sync_copy(blk_v, out_hbm.at[pl.ds(c * half * N2COL + w0, blk_words)])
      return 0
    lax.fori_loop(0, rpt // blk_rows, wr, 0)

  kern = pl.kernel(
      body,
      out_type=jax.ShapeDtypeStruct((N2PAD * N2COL,), jnp.float32),
      mesh=mesh,
      scratch_types=[
          pltpu.VMEM((CH,), jnp.int32),
          pltpu.VMEM((CH,), jnp.int32),
          pltpu.VMEM((CH,), jnp.float32),
          pltpu.VMEM((blk_words,), jnp.float32),
          pltpu.VMEM_SHARED(((N2PAD // 2) * N2COL,), jnp.float32),
      ],
  )
  return kern


def _tc1_body(xt_ref, w_ref, b_ref, o_ref):
  bn = xt_ref.shape[0]
  xb = xt_ref[...]
  x2 = xb.reshape(bn * NT * NB, KCH)
  acc = jnp.zeros((bn, HW, NB, NG1), jnp.float32)
  for h in range(HW):
    z = jnp.dot(x2, w_ref[h], preferred_element_type=jnp.float32)
    z = z.reshape(bn, NT, NB, NG1)
    acc = acc + z[:, h:h + HW, :, :]
  acc = acc + b_ref[0][None, None, None, :]
  o_ref[...] = jnp.maximum(acc, 0.0)


def _tc2_body(m_ref, x_ref, o_ref):
  kk = pl.program_id(1)

  @pl.when(kk == 0)
  def _():
    o_ref[...] = jnp.zeros_like(o_ref)

  o_ref[0] += jnp.dot(m_ref[...], x_ref[0], preferred_element_type=jnp.float32)


def _tc3_body(a_ref, x_ref, w_ref, b_ref, o_ref):
  a2 = a_ref[...]
  xkm1 = x_ref[0]
  out = jnp.dot(xkm1, w_ref[0], preferred_element_type=jnp.float32)
  xk = jnp.dot(a2, xkm1, preferred_element_type=jnp.float32)
  out = out + jnp.dot(xk, w_ref[1], preferred_element_type=jnp.float32)
  for k in range(2, KCH):
    xnew = 2.0 * jnp.dot(a2, xk, preferred_element_type=jnp.float32) - xkm1
    xkm1 = xk
    xk = xnew
    out = out + jnp.dot(xk, w_ref[k], preferred_element_type=jnp.float32)
  o_ref[0] = out + b_ref[0][None, :]


def _tc4_body(x_ref, w_ref, b_ref, o_ref):
  logits = jnp.dot(x_ref[...], w_ref[...], preferred_element_type=jnp.float32)
  logits = logits + b_ref[0][None, :]
  m = jnp.max(logits, axis=1, keepdims=True)
  ls = logits - m
  lse = jnp.log(jnp.sum(jnp.exp(ls), axis=1, keepdims=True))
  o_ref[...] = ls - lse


def _pad_edges(src, dst, w, e_pad, n_nodes):
  e = src.shape[0]
  extra = e_pad - e
  fill = (jnp.arange(extra, dtype=jnp.int32) % n_nodes)
  src_p = jnp.concatenate([src.astype(jnp.int32), fill])
  dst_p = jnp.concatenate([dst.astype(jnp.int32), fill])
  w_p = jnp.concatenate([w, jnp.zeros((extra,), jnp.float32)])
  return src_p, dst_p, w_p


def kernel(x, edge_index1, edge_weight1, edge_index2, edge_weight2, mapping,
           W1, b1, W2, b2, Wfc, bfc):
  xp = jnp.transpose(x, (1, 2, 0)).reshape(NN1, NT * NB)
  xp = jnp.pad(xp, ((0, N1P - NN1), (0, F1 - NT * NB)))
  y0 = jnp.transpose(xp.reshape(N1P, 2, F1H), (1, 0, 2)).reshape(2 * N1P, F1H)

  e1_pad = _ceil_to(edge_index1.shape[1], SC_TILES * CH)
  src1, dst1, w1 = _pad_edges(edge_index1[0], edge_index1[1], edge_weight1,
                              e1_pad, NN1)
  e2_pad = _ceil_to(edge_index2.shape[1], SC_TILES * CH)
  src2, dst2, w2 = _pad_edges(edge_index2[0], edge_index2[1], edge_weight2,
                              e2_pad, NN2)

  prop_first = _make_sc_prop(N1P, e1_pad, True)
  prop_rec = _make_sc_prop(N1P, e1_pad, False)

  w1_16 = jnp.broadcast_to(w1[:, None], (e1_pad, LANES))
  ys = [y0]
  dummy = y0
  y1 = prop_first(y0, src1, dst1, w1_16, dummy)
  ys.append(y1)
  for _ in range(2, KCH):
    ynew = prop_rec(ys[-1], src1, dst1, w1_16, ys[-2])
    ys.append(ynew)
  txs = jnp.stack(ys)

  txf = txs.reshape(KCH, 2, N1P, F1H).transpose(0, 2, 1, 3).reshape(KCH, N1P, F1)
  txf = txf[:, :NN1, :NT * NB].reshape(KCH, NN1, NT, NB).transpose(1, 2, 3, 0)

  w1h = jnp.transpose(W1[:, 0], (2, 1, 0))
  b1r = b1.reshape(1, NG1)
  bn = 50
  h1 = pl.pallas_call(
      _tc1_body,
      grid=(NN1 // bn,),
      in_specs=[
          pl.BlockSpec((bn, NT, NB, KCH), lambda i: (i, 0, 0, 0)),
          pl.BlockSpec((HW, KCH, NG1), lambda i: (0, 0, 0)),
          pl.BlockSpec((1, NG1), lambda i: (0, 0)),
      ],
      out_specs=pl.BlockSpec((bn, HW, NB, NG1), lambda i: (i, 0, 0, 0)),
      out_shape=jax.ShapeDtypeStruct((NN1, HW, NB, NG1), jnp.float32),
  )(txf, w1h, b1r)

  h1t = h1.transpose(2, 0, 3, 1).reshape(NB, NN1, NG1 * HW)

  bk = 1024
  n1p = _ceil_to(NN1, bk)
  map_p = jnp.pad(mapping, ((0, 0), (0, n1p - NN1)))
  h1t_p = jnp.pad(h1t, ((0, 0), (0, n1p - NN1), (0, 0)))
  xt2 = pl.pallas_call(
      _tc2_body,
      grid=(NB, n1p // bk),
      in_specs=[
          pl.BlockSpec((NN2, bk), lambda b, k: (0, k)),
          pl.BlockSpec((1, bk, NG1 * HW), lambda b, k: (b, k, 0)),
      ],
      out_specs=pl.BlockSpec((1, NN2, NG1 * HW), lambda b, k: (b, 0, 0)),
      out_shape=jax.ShapeDtypeStruct((NB, NN2, NG1 * HW), jnp.float32),
  )(map_p, h1t_p)

  dense_a2 = _make_sc_dense_a2(e2_pad)
  a2 = dense_a2(src2, dst2, w2).reshape(N2PAD, N2COL)
  a2 = a2[:, :NN2]
  a2 = jnp.pad(a2, ((0, 0), (0, N2PAD - NN2)))

  x0 = jnp.pad(xt2, ((0, 0), (0, N2PAD - NN2), (0, 0)))
  w2r = jnp.transpose(W2, (2, 1, 3, 0)).reshape(KCH, NG1 * HW, NG2)
  b2r = b2.reshape(1, NG2)
  out2 = pl.pallas_call(
      _tc3_body,
      grid=(NB,),
      in_specs=[
          pl.BlockSpec((N2PAD, N2PAD), lambda b: (0, 0)),
          pl.BlockSpec((1, N2PAD, NG1 * HW), lambda b: (b, 0, 0)),
          pl.BlockSpec((KCH, NG1 * HW, NG2), lambda b: (0, 0, 0)),
          pl.BlockSpec((1, NG2), lambda b: (0, 0)),
      ],
      out_specs=pl.BlockSpec((1, N2PAD, NG2), lambda b: (b, 0, 0)),
      out_shape=jax.ShapeDtypeStruct((NB, N2PAD, NG2), jnp.float32),
  )(a2, x0, w2r, b2r)

  flat = out2[:, :NN2, :].transpose(1, 2, 0).reshape(NB, NN2 * NG2)
  wfct = Wfc.T
  bfcr = bfc.reshape(1, NC)
  out = pl.pallas_call(
      _tc4_body,
      in_specs=[
          pl.BlockSpec((NB, NN2 * NG2), lambda: (0, 0)),
          pl.BlockSpec((NN2 * NG2, NC), lambda: (0, 0)),
          pl.BlockSpec((1, NC), lambda: (0, 0)),
      ],
      out_specs=pl.BlockSpec((NB, NC), lambda: (0, 0)),
      out_shape=jax.ShapeDtypeStruct((NB, NC), jnp.float32),
  )(flat, wfct, bfcr)
  return out


# speedup vs baseline: 41.5066x; 1.2372x over previous
import functools

import jax
import jax.numpy as jnp
from jax import lax
from jax.experimental import pallas as pl
from jax.experimental.pallas import tpu as pltpu
from jax.experimental.pallas import tpu_sc as plsc

KCH = 12
HW = 15
NG1 = 32
NG2 = 64
NN1 = 10000
N1P = 10240
NN2 = 1000
NT = 29
NB = 8
NC = 6

SC_CORES = 2
SC_TILES = 16
LANES = 16

F1 = 256
F1H = F1 // 2
CH = 64

N2PAD = 1024
N2COL = 1008


def _ceil_to(a, m):
  return (a + m - 1) // m * m


def _make_sc_prop(n_nodes, e_pad, first):
  cpt = e_pad // (SC_TILES * CH)
  rpt = n_nodes // SC_TILES
  rb = 32
  assert rpt % rb == 0

  mesh = plsc.VectorSubcoreMesh(core_axis_name="c", subcore_axis_name="s")

  assert cpt % 2 == 0

  def body(y_hbm, src_hbm, dst_hbm, w16_hbm, prev2_hbm, out_hbm,
           si_v, di_v, w16_v, rows_v, blk_v, prev_v, acc_sh,
           rs0, rs1, gs0, gs1):
    c = lax.axis_index("c")
    s = lax.axis_index("s")
    rsem = [rs0, rs1]
    gsem = [gs0, gs1]

    for i in range(rb):
      for f in range(F1H // LANES):
        blk_v[i, pl.ds(f * LANES, LANES)] = jnp.zeros((LANES,), jnp.float32)
    def zero_blk(t, _):
      pltpu.sync_copy(blk_v, acc_sh.at[pl.ds(s * rpt + t * rb, rb)])
      return 0
    lax.fori_loop(0, rpt // rb, zero_blk, 0)
    plsc.subcore_barrier()

    def rec_copies(cj, slot):
      base = (s * cpt + cj) * CH
      return (
          pltpu.make_async_copy(src_hbm.at[pl.ds(base, CH)], si_v.at[slot],
                                rsem[slot]),
          pltpu.make_async_copy(dst_hbm.at[pl.ds(base, CH)], di_v.at[slot],
                                rsem[slot]),
          pltpu.make_async_copy(w16_hbm.at[pl.ds(base, CH)], w16_v.at[slot],
                                rsem[slot]),
      )

    def issue_rec(cj, slot):
      for d in rec_copies(cj, slot):
        d.start()

    def wait_rec(cj, slot):
      for d in rec_copies(cj, slot):
        d.wait()

    def gather_copy(slot):
      return pltpu.make_async_copy(y_hbm.at[si_v.at[slot]], rows_v.at[slot],
                                   gsem[slot])

    def prep_gather(slot):
      off = c * n_nodes
      for i in range(CH // LANES):
        sl = pl.ds(i * LANES, LANES)
        si_v[slot, sl] = si_v[slot, sl] + off
      gather_copy(slot).start()

    issue_rec(0, 0)
    issue_rec(1, 1)
    wait_rec(0, 0)
    prep_gather(0)

    def duo(t, _):
      for b in range(2):
        j = 2 * t + b
        q = b ^ 1

        @pl.when(j + 1 < cpt)
        def _():
          wait_rec(j + 1, q)
          prep_gather(q)

        gather_copy(b).wait()
        for e in range(CH):
          wb = w16_v[b, e]
          for f in range(F1H // LANES):
            sl = pl.ds(f * LANES, LANES)
            rows_v[b, e, sl] = rows_v[b, e, sl] * wb
        pltpu.sync_copy(rows_v.at[b], acc_sh.at[di_v.at[b]], add=True)

        @pl.when(j + 2 < cpt)
        def _():
          issue_rec(j + 2, b)
      return 0
    lax.fori_loop(0, cpt // 2, duo, 0)
    plsc.subcore_barrier()

    def epi(t, _):
      r0 = s * rpt + t * rb
      pltpu.sync_copy(acc_sh.at[pl.ds(r0, rb)], blk_v)
      if not first:
        pltpu.sync_copy(prev2_hbm.at[pl.ds(c * n_nodes + r0, rb)], prev_v)
        for i in range(rb):
          for f in range(F1H // LANES):
            sl = pl.ds(f * LANES, LANES)
            a = blk_v[i, sl]
            blk_v[i, sl] = a + a - prev_v[i, sl]
      pltpu.sync_copy(blk_v, out_hbm.at[pl.ds(c * n_nodes + r0, rb)])
      return 0
    lax.fori_loop(0, rpt // rb, epi, 0)

  kern = pl.kernel(
      body,
      out_type=jax.ShapeDtypeStruct((2 * n_nodes, F1H), jnp.float32),
      mesh=mesh,
      scratch_types=[
          pltpu.VMEM((2, CH), jnp.int32),
          pltpu.VMEM((2, CH), jnp.int32),
          pltpu.VMEM((2, CH, LANES), jnp.float32),
          pltpu.VMEM((2, CH, F1H), jnp.float32),
          pltpu.VMEM((rb, F1H), jnp.float32),
          pltpu.VMEM((rb, F1H), jnp.float32),
          pltpu.VMEM_SHARED((n_nodes, F1H), jnp.float32),
          pltpu.SemaphoreType.DMA,
          pltpu.SemaphoreType.DMA,
          pltpu.SemaphoreType.DMA,
          pltpu.SemaphoreType.DMA,
      ],
  )
  return kern


def _make_sc_dense_a2(e_pad):
  cpt = e_pad // (SC_TILES * CH)
  half = N2PAD // 2
  rpt = half // SC_TILES
  mesh = plsc.VectorSubcoreMesh(core_axis_name="c", subcore_axis_name="s")

  blk_rows = 8
  blk_words = blk_rows * N2COL

  def body(src_hbm, dst_hbm, w_hbm, out_hbm, idx_v, src_v, w_v, blk_v, a_sh):
    c = lax.axis_index("c")
    s = lax.axis_index("s")

    for i in range(blk_words // LANES):
      blk_v[pl.ds(i * LANES, LANES)] = jnp.zeros((LANES,), jnp.float32)
    def zero_blk(t, _):
      pltpu.sync_copy(blk_v, a_sh.at[pl.ds((s * rpt + t * blk_rows) * N2COL,
                                           blk_words)])
      return 0
    lax.fori_loop(0, rpt // blk_rows, zero_blk, 0)
    plsc.subcore_barrier()

    def chunk(j, _):
      base = (s * cpt + j) * CH
      pltpu.sync_copy(dst_hbm.at[pl.ds(base, CH)], idx_v)
      pltpu.sync_copy(src_hbm.at[pl.ds(base, CH)], src_v)
      pltpu.sync_copy(w_hbm.at[pl.ds(base, CH)], w_v)
      lo = c * half
      for i in range(CH // LANES):
        sl = pl.ds(i * LANES, LANES)
        d = idx_v[sl]
        ok = (d >= lo) & (d < lo + half)
        spread = lax.iota(jnp.int32, LANES) + i * LANES
        flat = (d - lo) * N2COL + src_v[sl]
        idx_v[sl] = jnp.where(ok, flat, spread)
        w_v[sl] = jnp.where(ok, w_v[sl], jnp.zeros((LANES,), jnp.float32))
      pltpu.sync_copy(w_v, a_sh.at[idx_v], add=True)
      return 0
    lax.fori_loop(0, cpt, chunk, 0)
    plsc.subcore_barrier()

    def wr(t, _):
      w0 = (s * rpt + t * blk_rows) * N2COL
      pltpu.sync_copy(a_sh.at[pl.ds(w0, blk_words)], blk_v)
      pltpu.sync_copy(blk_v, out_hbm.at[pl.ds(c * half * N2COL + w0, blk_words)])
      return 0
    lax.fori_loop(0, rpt // blk_rows, wr, 0)

  kern = pl.kernel(
      body,
      out_type=jax.ShapeDtypeStruct((N2PAD * N2COL,), jnp.float32),
      mesh=mesh,
      scratch_types=[
          pltpu.VMEM((CH,), jnp.int32),
          pltpu.VMEM((CH,), jnp.int32),
          pltpu.VMEM((CH,), jnp.float32),
          pltpu.VMEM((blk_words,), jnp.float32),
          pltpu.VMEM_SHARED(((N2PAD // 2) * N2COL,), jnp.float32),
      ],
  )
  return kern


def _tc1_body(xt_ref, w_ref, b_ref, o_ref):
  bn = xt_ref.shape[0]
  xb = xt_ref[...]
  x2 = xb.reshape(bn * NT * NB, KCH)
  acc = jnp.zeros((bn, HW, NB, NG1), jnp.float32)
  for h in range(HW):
    z = jnp.dot(x2, w_ref[h], preferred_element_type=jnp.float32)
    z = z.reshape(bn, NT, NB, NG1)
    acc = acc + z[:, h:h + HW, :, :]
  acc = acc + b_ref[0][None, None, None, :]
  o_ref[...] = jnp.maximum(acc, 0.0)


def _tc2_body(m_ref, x_ref, o_ref):
  kk = pl.program_id(1)

  @pl.when(kk == 0)
  def _():
    o_ref[...] = jnp.zeros_like(o_ref)

  o_ref[0] += jnp.dot(m_ref[...], x_ref[0], preferred_element_type=jnp.float32)


def _tc3_body(a_ref, x_ref, w_ref, b_ref, o_ref):
  a2 = a_ref[...]
  xkm1 = x_ref[0]
  out = jnp.dot(xkm1, w_ref[0], preferred_element_type=jnp.float32)
  xk = jnp.dot(a2, xkm1, preferred_element_type=jnp.float32)
  out = out + jnp.dot(xk, w_ref[1], preferred_element_type=jnp.float32)
  for k in range(2, KCH):
    xnew = 2.0 * jnp.dot(a2, xk, preferred_element_type=jnp.float32) - xkm1
    xkm1 = xk
    xk = xnew
    out = out + jnp.dot(xk, w_ref[k], preferred_element_type=jnp.float32)
  o_ref[0] = out + b_ref[0][None, :]


def _tc4_body(x_ref, w_ref, b_ref, o_ref):
  logits = jnp.dot(x_ref[...], w_ref[...], preferred_element_type=jnp.float32)
  logits = logits + b_ref[0][None, :]
  m = jnp.max(logits, axis=1, keepdims=True)
  ls = logits - m
  lse = jnp.log(jnp.sum(jnp.exp(ls), axis=1, keepdims=True))
  o_ref[...] = ls - lse


def _pad_edges(src, dst, w, e_pad, n_nodes):
  e = src.shape[0]
  extra = e_pad - e
  fill = (jnp.arange(extra, dtype=jnp.int32) % n_nodes)
  src_p = jnp.concatenate([src.astype(jnp.int32), fill])
  dst_p = jnp.concatenate([dst.astype(jnp.int32), fill])
  w_p = jnp.concatenate([w, jnp.zeros((extra,), jnp.float32)])
  return src_p, dst_p, w_p


def kernel(x, edge_index1, edge_weight1, edge_index2, edge_weight2, mapping,
           W1, b1, W2, b2, Wfc, bfc):
  xp = jnp.transpose(x, (1, 2, 0)).reshape(NN1, NT * NB)
  xp = jnp.pad(xp, ((0, N1P - NN1), (0, F1 - NT * NB)))
  y0 = jnp.transpose(xp.reshape(N1P, 2, F1H), (1, 0, 2)).reshape(2 * N1P, F1H)

  e1_pad = _ceil_to(edge_index1.shape[1], SC_TILES * CH * 2)
  src1, dst1, w1 = _pad_edges(edge_index1[0], edge_index1[1], edge_weight1,
                              e1_pad, NN1)
  e2_pad = _ceil_to(edge_index2.shape[1], SC_TILES * CH)
  src2, dst2, w2 = _pad_edges(edge_index2[0], edge_index2[1], edge_weight2,
                              e2_pad, NN2)

  prop_first = _make_sc_prop(N1P, e1_pad, True)
  prop_rec = _make_sc_prop(N1P, e1_pad, False)

  w1_16 = jnp.broadcast_to(w1[:, None], (e1_pad, LANES))
  ys = [y0]
  dummy = y0
  y1 = prop_first(y0, src1, dst1, w1_16, dummy)
  ys.append(y1)
  for _ in range(2, KCH):
    ynew = prop_rec(ys[-1], src1, dst1, w1_16, ys[-2])
    ys.append(ynew)
  txs = jnp.stack(ys)

  txf = txs.reshape(KCH, 2, N1P, F1H).transpose(0, 2, 1, 3).reshape(KCH, N1P, F1)
  txf = txf[:, :NN1, :NT * NB].reshape(KCH, NN1, NT, NB).transpose(1, 2, 3, 0)

  w1h = jnp.transpose(W1[:, 0], (2, 1, 0))
  b1r = b1.reshape(1, NG1)
  bn = 50
  h1 = pl.pallas_call(
      _tc1_body,
      grid=(NN1 // bn,),
      in_specs=[
          pl.BlockSpec((bn, NT, NB, KCH), lambda i: (i, 0, 0, 0)),
          pl.BlockSpec((HW, KCH, NG1), lambda i: (0, 0, 0)),
          pl.BlockSpec((1, NG1), lambda i: (0, 0)),
      ],
      out_specs=pl.BlockSpec((bn, HW, NB, NG1), lambda i: (i, 0, 0, 0)),
      out_shape=jax.ShapeDtypeStruct((NN1, HW, NB, NG1), jnp.float32),
  )(txf, w1h, b1r)

  h1t = h1.transpose(2, 0, 3, 1).reshape(NB, NN1, NG1 * HW)

  bk = 1024
  n1p = _ceil_to(NN1, bk)
  map_p = jnp.pad(mapping, ((0, 0), (0, n1p - NN1)))
  h1t_p = jnp.pad(h1t, ((0, 0), (0, n1p - NN1), (0, 0)))
  xt2 = pl.pallas_call(
      _tc2_body,
      grid=(NB, n1p // bk),
      in_specs=[
          pl.BlockSpec((NN2, bk), lambda b, k: (0, k)),
          pl.BlockSpec((1, bk, NG1 * HW), lambda b, k: (b, k, 0)),
      ],
      out_specs=pl.BlockSpec((1, NN2, NG1 * HW), lambda b, k: (b, 0, 0)),
      out_shape=jax.ShapeDtypeStruct((NB, NN2, NG1 * HW), jnp.float32),
  )(map_p, h1t_p)

  dense_a2 = _make_sc_dense_a2(e2_pad)
  a2 = dense_a2(src2, dst2, w2).reshape(N2PAD, N2COL)
  a2 = a2[:, :NN2]
  a2 = jnp.pad(a2, ((0, 0), (0, N2PAD - NN2)))

  x0 = jnp.pad(xt2, ((0, 0), (0, N2PAD - NN2), (0, 0)))
  w2r = jnp.transpose(W2, (2, 1, 3, 0)).reshape(KCH, NG1 * HW, NG2)
  b2r = b2.reshape(1, NG2)
  out2 = pl.pallas_call(
      _tc3_body,
      grid=(NB,),
      in_specs=[
          pl.BlockSpec((N2PAD, N2PAD), lambda b: (0, 0)),
          pl.BlockSpec((1, N2PAD, NG1 * HW), lambda b: (b, 0, 0)),
          pl.BlockSpec((KCH, NG1 * HW, NG2), lambda b: (0, 0, 0)),
          pl.BlockSpec((1, NG2), lambda b: (0, 0)),
      ],
      out_specs=pl.BlockSpec((1, N2PAD, NG2), lambda b: (b, 0, 0)),
      out_shape=jax.ShapeDtypeStruct((NB, N2PAD, NG2), jnp.float32),
  )(a2, x0, w2r, b2r)

  flat = out2[:, :NN2, :].transpose(1, 2, 0).reshape(NB, NN2 * NG2)
  wfct = Wfc.T
  bfcr = bfc.reshape(1, NC)
  out = pl.pallas_call(
      _tc4_body,
      in_specs=[
          pl.BlockSpec((NB, NN2 * NG2), lambda: (0, 0)),
          pl.BlockSpec((NN2 * NG2, NC), lambda: (0, 0)),
          pl.BlockSpec((1, NC), lambda: (0, 0)),
      ],
      out_specs=pl.BlockSpec((NB, NC), lambda: (0, 0)),
      out_shape=jax.ShapeDtypeStruct((NB, NC), jnp.float32),
  )(flat, wfct, bfcr)
  return out
